# baseline (speedup 1.0000x reference)
import functools

import jax
import jax.numpy as jnp
from jax import lax
from jax.experimental import pallas as pl
from jax.experimental.pallas import tpu as pltpu
from jax.experimental.pallas import tpu_sc as plsc

N_LAYERS = 3
NC = 2
NS = 16
NW = NC * NS
C = 400
K = 128


def _scal(ref, i):
    return ref[pl.ds(i, 16)][0]


def _make_spmm(n, d, e_pad, nchunk):
    ch_per_w = (nchunk + NW - 1) // NW
    nd = d // 16
    mesh = plsc.VectorSubcoreMesh(core_axis_name="c", subcore_axis_name="s")

    @functools.partial(
        pl.kernel,
        out_type=jax.ShapeDtypeStruct((n * d,), jnp.float32),
        mesh=mesh,
        scratch_types=[
            pltpu.VMEM((C * d,), jnp.float32),
            pltpu.VMEM((K, d), jnp.float32),
            pltpu.VMEM((K,), jnp.int32),
            pltpu.VMEM((K + 16,), jnp.float32),
            pltpu.VMEM((K + 16,), jnp.int32),
            pltpu.VMEM((C + 32,), jnp.int32),
            pltpu.SemaphoreType.DMA,
        ],
    )
    def spmm(x_hbm, cols_hbm, vals_hbm, rows_hbm, off_hbm, y_hbm,
             acc, gbuf, cbuf, vbuf, rbuf, robuf, sem):
        wid = lax.axis_index("s") * NC + lax.axis_index("c")
        zero16 = jnp.zeros((16,), jnp.float32)
        zeros8 = (zero16,) * nd

        def process_chunk(chunk):
            r0 = chunk * C
            pltpu.sync_copy(off_hbm.at[pl.ds(r0, C + 16)],
                            robuf.at[pl.ds(0, C + 16)])
            e_lo = _scal(robuf, 0)
            e_hi = _scal(robuf, C)

            def zero_body(i, _):
                acc[pl.ds(i * 16, 16)] = zero16
                return 0
            lax.fori_loop(0, C * d // 16, zero_body, 0)

            e_al = (e_lo // 8) * 8
            nb = (e_hi - e_al + K - 1) // K

            def batch_body(b, _):
                base = e_al + b * K
                pltpu.sync_copy(cols_hbm.at[pl.ds(base, K)], cbuf)
                pltpu.sync_copy(vals_hbm.at[pl.ds(base, K)], vbuf.at[pl.ds(0, K)])
                pltpu.sync_copy(rows_hbm.at[pl.ds(base, K)], rbuf.at[pl.ds(0, K)])
                pltpu.async_copy(x_hbm.at[cbuf], gbuf, sem).wait()
                lo = jnp.maximum(e_lo - base, 0)
                hi = jnp.minimum(e_hi - base, K)

                @pl.when(hi > lo)
                def _():
                    rlo = _scal(rbuf, lo)
                    rhi = _scal(rbuf, hi - 1)

                    def row_body(r, _):
                        j = r - r0
                        es = jnp.maximum(_scal(robuf, j) - base, lo)
                        ee = jnp.minimum(_scal(robuf, j + 1) - base, hi)

                        def edge_sub(e, a):
                            vv = vbuf[pl.ds(e, 16)][0]
                            return tuple(
                                a[db] + vv * gbuf[e, pl.ds(db * 16, 16)]
                                for db in range(nd))
                        a = lax.fori_loop(es, ee, edge_sub, zeros8)
                        rb = j * d
                        for db in range(nd):
                            sl = pl.ds(rb + db * 16, 16)
                            acc[sl] = acc[sl] + a[db]
                        return 0
                    lax.fori_loop(rlo, rhi + 1, row_body, 0)
                return 0
            lax.fori_loop(0, nb, batch_body, 0)
            pltpu.sync_copy(acc, y_hbm.at[pl.ds(r0 * d, C * d)])

        for t in range(ch_per_w):
            chunk = wid + t * NW
            if (t + 1) * NW <= nchunk:
                process_chunk(chunk)
            else:
                @pl.when(chunk < nchunk)
                def _():
                    process_chunk(chunk)

    return spmm


def _norm_acc_kernel(w, y_ref, res_ref, x_ref, out_ref):
    y = y_ref[...]
    ss = jnp.sum(y * y, axis=1, keepdims=True)
    inv = lax.rsqrt(jnp.maximum(ss, 1e-24))
    x = y * inv
    x_ref[...] = x
    out_ref[...] = res_ref[...] + x * w


def _make_norm(n, d, w):
    br = 400
    grid = n // br
    return pl.pallas_call(
        functools.partial(_norm_acc_kernel, w),
        grid=(grid,),
        in_specs=[
            pl.BlockSpec((br, d), lambda i: (i, 0)),
            pl.BlockSpec((br, d), lambda i: (i, 0)),
        ],
        out_specs=[
            pl.BlockSpec((br, d), lambda i: (i, 0)),
            pl.BlockSpec((br, d), lambda i: (i, 0)),
        ],
        out_shape=[
            jax.ShapeDtypeStruct((n, d), jnp.float32),
            jax.ShapeDtypeStruct((n, d), jnp.float32),
        ],
    )


def kernel(in_embs, edge_vals, edge_rows, edge_cols):
    n, d = in_embs.shape
    e = edge_rows.shape[0]
    assert n % C == 0
    nchunk = n // C
    e_pad = (e // K + 2) * K

    boundaries = jnp.arange(n + 1, dtype=jnp.int32)
    off = jnp.searchsorted(edge_rows, boundaries, side="left").astype(jnp.int32)
    off = jnp.pad(off, (0, 31), mode="edge")
    cols_p = jnp.pad(edge_cols, (0, e_pad - e))
    vals_p = jnp.pad(edge_vals, (0, e_pad - e))
    rows_p = jnp.pad(edge_rows, (0, e_pad - e))

    spmm = _make_spmm(n, d, e_pad, nchunk)

    res = in_embs
    x = in_embs
    for i in range(N_LAYERS):
        y = spmm(x, cols_p, vals_p, rows_p, off).reshape(n, d)
        x, res = _make_norm(n, d, 1.0 / (i + 1))(y, res)
    return res


# speedup vs baseline: 9.8359x; 9.8359x over previous
import functools

import jax
import jax.numpy as jnp
from jax import lax
from jax.experimental import pallas as pl
from jax.experimental.pallas import tpu as pltpu
from jax.experimental.pallas import tpu_sc as plsc

N_LAYERS = 3
NC = 2
NS = 16
NW = NC * NS
C = 400
CP = C + 8
K = 128
ZR = 51


def _scal(ref, i):
    return ref[pl.ds(i, 16)][0]


def _make_spmm(n, d, e_pad, nchunk, noff_pad):
    ch_per_w = (nchunk + NW - 1) // NW
    nd = d // 16
    ng = K // 16
    mesh = plsc.VectorSubcoreMesh(core_axis_name="c", subcore_axis_name="s")

    @functools.partial(
        pl.kernel,
        out_type=jax.ShapeDtypeStruct((n, d), jnp.float32),
        mesh=mesh,
        compiler_params=pltpu.CompilerParams(needs_layout_passes=False),
        scratch_types=[
            pltpu.VMEM_SHARED((NS * CP, d), jnp.float32),
            pltpu.VMEM((K, d), jnp.float32),
            pltpu.VMEM((K, d), jnp.float32),
            pltpu.VMEM((K, d), jnp.float32),
            pltpu.VMEM((K, d), jnp.float32),
            pltpu.VMEM((K,), jnp.int32),
            pltpu.VMEM((K,), jnp.int32),
            pltpu.VMEM((K,), jnp.float32),
            pltpu.VMEM((K,), jnp.float32),
            pltpu.VMEM((K,), jnp.int32),
            pltpu.VMEM((K,), jnp.int32),
            pltpu.VMEM((K,), jnp.int32),
            pltpu.VMEM((K,), jnp.int32),
            pltpu.VMEM((noff_pad + 16,), jnp.int32),
            pltpu.VMEM((ZR, d), jnp.float32),
            pltpu.SemaphoreType.DMA,
            pltpu.SemaphoreType.DMA,
            pltpu.SemaphoreType.DMA,
            pltpu.SemaphoreType.DMA,
            pltpu.SemaphoreType.DMA,
            pltpu.SemaphoreType.DMA,
        ],
    )
    def spmm(x_hbm, cols_hbm, vals_hbm, rows_hbm, offc_hbm, y_hbm,
             acc, gbuf0, gbuf1, sbuf0, sbuf1, cbuf0, cbuf1, vbuf0, vbuf1,
             rbuf0, rbuf1, ribuf0, ribuf1, ocbuf, zbuf,
             semi0, semi1, semg0, semg1, sems0, sems1):
        sid = lax.axis_index("s")
        wid = sid * NC + lax.axis_index("c")
        sbase = sid * CP
        gbuf = (gbuf0, gbuf1)
        sbuf = (sbuf0, sbuf1)
        cbuf = (cbuf0, cbuf1)
        vbuf = (vbuf0, vbuf1)
        rbuf = (rbuf0, rbuf1)
        ribuf = (ribuf0, ribuf1)
        semi = (semi0, semi1)
        semg = (semg0, semg1)
        sems = (sems0, sems1)
        iota16 = lax.iota(jnp.int32, 16)

        pltpu.sync_copy(offc_hbm, ocbuf.at[pl.ds(0, noff_pad)])

        def zrow(i, _):
            for db in range(nd):
                zbuf[i, pl.ds(db * 16, 16)] = jnp.zeros((16,), jnp.float32)
            return 0
        lax.fori_loop(0, ZR, zrow, 0)

        def process_chunk(chunk):
            r0 = chunk * C
            e_lo = _scal(ocbuf, chunk)
            e_hi = _scal(ocbuf, chunk + 1)
            e_al = (e_lo // 8) * 8
            nb = (e_hi - e_al + K - 1) // K

            for i in range(CP // ZR):
                pltpu.sync_copy(
                    zbuf.at[pl.ds(0, ZR), :],
                    acc.at[pl.ds(sbase + i * ZR, ZR), :])

            def issue_idx(b, j):
                base = e_al + b * K
                pltpu.async_copy(cols_hbm.at[pl.ds(base, K)], cbuf[j], semi[j])
                pltpu.async_copy(vals_hbm.at[pl.ds(base, K)], vbuf[j], semi[j])
                pltpu.async_copy(rows_hbm.at[pl.ds(base, K)], rbuf[j], semi[j])

            def wait_idx(j):
                pltpu.make_async_copy(cols_hbm.at[pl.ds(0, K)], cbuf[j], semi[j]).wait()
                pltpu.make_async_copy(vals_hbm.at[pl.ds(0, K)], vbuf[j], semi[j]).wait()
                pltpu.make_async_copy(rows_hbm.at[pl.ds(0, K)], rbuf[j], semi[j]).wait()

            def issue_gather(j):
                pltpu.async_copy(x_hbm.at[cbuf[j]], gbuf[j], semg[j])

            def wait_gather(j):
                pltpu.make_async_copy(x_hbm.at[cbuf[j]], gbuf[j], semg[j]).wait()

            def issue_scatter(j):
                pltpu.async_copy(sbuf[j], acc.at[ribuf[j]], sems[j], add=True)

            def wait_scatter(j):
                pltpu.make_async_copy(sbuf[j], acc.at[ribuf[j]], sems[j]).wait()

            def scale(j):
                gb, sb, vb, rb, rib = gbuf[j], sbuf[j], vbuf[j], rbuf[j], ribuf[j]
                for g in range(ng):
                    row16 = rb[pl.ds(g * 16, 16)]
                    rloc = row16 - r0
                    ok = (rloc >= 0) & (rloc < C)
                    rib[pl.ds(g * 16, 16)] = jnp.where(ok, rloc, C) + sbase

                def trip(it, _):
                    e0 = it * 8
                    for u in range(8):
                        e = e0 + u
                        vbc = plsc.load_gather(vb, [jnp.full((16,), e, jnp.int32)])
                        for db in range(nd):
                            sl = pl.ds(db * 16, 16)
                            sb[e, sl] = vbc * gb[e, sl]
                    return 0
                lax.fori_loop(0, K // 8, trip, 0)

            @pl.when(nb > 0)
            def _():
                issue_idx(0, 0)
                wait_idx(0)
                issue_gather(0)

            @pl.when(nb > 1)
            def _():
                issue_idx(1, 1)

            def pair_body(p, _):
                for j in (0, 1):
                    b = p * 2 + j

                    @pl.when(b < nb)
                    def _():
                        @pl.when(b + 1 < nb)
                        def _():
                            wait_idx(1 - j)
                            issue_gather(1 - j)
                        wait_gather(j)

                        @pl.when(b >= 2)
                        def _():
                            wait_scatter(j)
                        scale(j)
                        issue_scatter(j)

                        @pl.when(b + 2 < nb)
                        def _():
                            issue_idx(b + 2, j)
                return 0
            lax.fori_loop(0, (nb + 1) // 2, pair_body, 0)

            @pl.when(nb >= 2)
            def _():
                wait_scatter(0)
                wait_scatter(1)

            @pl.when(nb == 1)
            def _():
                wait_scatter(0)

            pltpu.sync_copy(acc.at[pl.ds(sbase, C), :],
                            y_hbm.at[pl.ds(r0, C), :])

        for t in range(ch_per_w):
            chunk = wid + t * NW
            if (t + 1) * NW <= nchunk:
                process_chunk(chunk)
            else:
                @pl.when(chunk < nchunk)
                def _():
                    process_chunk(chunk)

    return spmm


def _norm_acc_kernel(w, y_ref, res_ref, x_ref, out_ref):
    y = y_ref[...]
    ss = jnp.sum(y * y, axis=1, keepdims=True)
    inv = lax.rsqrt(jnp.maximum(ss, 1e-24))
    x = y * inv
    x_ref[...] = x
    out_ref[...] = res_ref[...] + x * w


def _make_norm(n, d, w):
    br = 400
    grid = n // br
    return pl.pallas_call(
        functools.partial(_norm_acc_kernel, w),
        grid=(grid,),
        in_specs=[
            pl.BlockSpec((br, d), lambda i: (i, 0)),
            pl.BlockSpec((br, d), lambda i: (i, 0)),
        ],
        out_specs=[
            pl.BlockSpec((br, d), lambda i: (i, 0)),
            pl.BlockSpec((br, d), lambda i: (i, 0)),
        ],
        out_shape=[
            jax.ShapeDtypeStruct((n, d), jnp.float32),
            jax.ShapeDtypeStruct((n, d), jnp.float32),
        ],
    )


def kernel(in_embs, edge_vals, edge_rows, edge_cols):
    n, d = in_embs.shape
    e = edge_rows.shape[0]
    assert n % C == 0
    nchunk = n // C
    noff_pad = ((nchunk + 1 + 15) // 16) * 16
    e_pad = (e // K + 2) * K

    boundaries = jnp.arange(nchunk + 1, dtype=jnp.int32) * C
    off = jnp.searchsorted(edge_rows, boundaries, side="left").astype(jnp.int32)
    off = jnp.pad(off, (0, noff_pad - (nchunk + 1)), mode="edge")
    cols_p = jnp.pad(edge_cols, (0, e_pad - e))
    vals_p = jnp.pad(edge_vals, (0, e_pad - e))
    rows_p = jnp.pad(edge_rows, (0, e_pad - e))

    spmm = _make_spmm(n, d, e_pad, nchunk, noff_pad)

    res = in_embs
    x = in_embs
    for i in range(N_LAYERS):
        y = spmm(x, cols_p, vals_p, rows_p, off)
        x, res = _make_norm(n, d, 1.0 / (i + 1))(y, res)
    return res
